# baseline (speedup 1.0000x reference)
import functools

import jax
import jax.numpy as jnp
from jax import lax
from jax.experimental import pallas as pl
from jax.experimental.pallas import tpu as pltpu
from jax.experimental.pallas import tpu_sc as plsc

_BATCH = 4
_SEQ = 8192
_B = _BATCH * _SEQ
_D = 64
_NC, _NS = 2, 16
_NW = _NC * _NS
_BPW = _B // _NW
_WPB = _SEQ // _BPW

_CH = 128
_NCH = _BPW // _CH
_NBUF = 4
_DEPTH = 2

_mesh = plsc.VectorSubcoreMesh(core_axis_name="c", subcore_axis_name="s")


@functools.partial(
    pl.kernel,
    mesh=_mesh,
    out_type=jax.ShapeDtypeStruct((_BATCH, _SEQ, 2 * _D), jnp.float32),
    scratch_types=[
        pltpu.VMEM((_BPW,), jnp.int32),
        [pltpu.VMEM((_CH, 2 * _D), jnp.float32) for _ in range(_NBUF)],
        pltpu.SemaphoreType.DMA((_NBUF,)),
        pltpu.SemaphoreType.DMA((_NBUF,)),
    ],
)
def _gather_rows(pos_hbm, ctab_hbm, out_hbm, idx_v, bufs, g_sem, w_sem):
    wid = lax.axis_index("s") * _NC + lax.axis_index("c")
    row = wid // _WPB
    off = (wid % _WPB) * _BPW
    pltpu.sync_copy(pos_hbm.at[row, pl.ds(off, _BPW)], idx_v)

    def start_gather(t):
        idx_sl = idx_v.at[pl.ds(t * _CH, _CH)]
        return pltpu.async_copy(ctab_hbm.at[idx_sl],
                                bufs[t % _NBUF], g_sem.at[t % _NBUF])

    def start_wb(t):
        dst = out_hbm.at[row, pl.ds(off + t * _CH, _CH)]
        return pltpu.async_copy(bufs[t % _NBUF], dst, w_sem.at[t % _NBUF])

    gathers = {t: start_gather(t) for t in range(_DEPTH)}
    wbs = {}
    for w in range(_NCH):
        nx = w + _DEPTH
        if nx < _NCH:
            if nx >= _NBUF:
                wbs[nx - _NBUF].wait()
            gathers[nx] = start_gather(nx)
        gathers[w].wait()
        wbs[w] = start_wb(w)
    for t in range(max(0, _NCH - _NBUF), _NCH):
        wbs[t].wait()


_TCROWS = 2048


def _split_body(c_ref, s_ref, co_ref):
    s_ref[...] = c_ref[:, :, :_D]
    co_ref[...] = c_ref[:, :, _D:]


_split = pl.pallas_call(
    _split_body,
    grid=(_BATCH, _SEQ // _TCROWS),
    in_specs=[pl.BlockSpec((1, _TCROWS, 2 * _D), lambda b, i: (b, i, 0))],
    out_specs=[
        pl.BlockSpec((1, _TCROWS, _D), lambda b, i: (b, i, 0)),
        pl.BlockSpec((1, _TCROWS, _D), lambda b, i: (b, i, 0)),
    ],
    out_shape=[
        jax.ShapeDtypeStruct((_BATCH, _SEQ, _D), jnp.float32),
        jax.ShapeDtypeStruct((_BATCH, _SEQ, _D), jnp.float32),
    ],
    compiler_params=pltpu.CompilerParams(
        dimension_semantics=("arbitrary", "arbitrary"),
    ),
)


def kernel(position, sin_values, cos_values):
    ctab = jnp.concatenate([sin_values, cos_values], axis=1)
    combined = _gather_rows(position, ctab)
    return tuple(_split(combined))


# speedup vs baseline: 1.2236x; 1.2236x over previous
import functools

import jax
import jax.numpy as jnp
from jax import lax
from jax.experimental import pallas as pl
from jax.experimental.pallas import tpu as pltpu
from jax.experimental.pallas import tpu_sc as plsc

_BATCH = 4
_SEQ = 8192
_B = _BATCH * _SEQ
_D = 64
_NC, _NS = 2, 16
_NW = _NC * _NS
_BPW = _B // _NW
_WPB = _SEQ // _BPW

_CH = 128
_NCH = _BPW // _CH
_NG = 2
_NSP = 2
_L = 16

_mesh = plsc.VectorSubcoreMesh(core_axis_name="c", subcore_axis_name="s")


@functools.partial(
    pl.kernel,
    mesh=_mesh,
    out_type=(
        jax.ShapeDtypeStruct((_BATCH, _SEQ, _D), jnp.float32),
        jax.ShapeDtypeStruct((_BATCH, _SEQ, _D), jnp.float32),
    ),
    scratch_types=[
        pltpu.VMEM((_BPW,), jnp.int32),
        [pltpu.VMEM((_CH, 2 * _D), jnp.float32) for _ in range(_NG)],
        [pltpu.VMEM((_CH, _D), jnp.float32) for _ in range(_NSP)],
        [pltpu.VMEM((_CH, _D), jnp.float32) for _ in range(_NSP)],
        pltpu.SemaphoreType.DMA((_NG,)),
        pltpu.SemaphoreType.DMA((2, _NSP)),
    ],
)
def _gather_rows(pos_hbm, ctab_hbm, out_sin, out_cos,
                 idx_v, gbufs, sbufs, cbufs, g_sem, w_sem):
    wid = lax.axis_index("s") * _NC + lax.axis_index("c")
    row = wid // _WPB
    off = (wid % _WPB) * _BPW
    pltpu.sync_copy(pos_hbm.at[row, pl.ds(off, _BPW)], idx_v)

    def start_gather(t):
        idx_sl = idx_v.at[pl.ds(t * _CH, _CH)]
        return pltpu.async_copy(ctab_hbm.at[idx_sl],
                                gbufs[t % _NG], g_sem.at[t % _NG])

    def deinterleave(t):
        gb, sb, cb = gbufs[t % _NG], sbufs[t % _NSP], cbufs[t % _NSP]

        def body(r, _):
            for j in range(_D // _L):
                sb[r, pl.ds(_L * j, _L)] = gb[r, pl.ds(_L * j, _L)]
                cb[r, pl.ds(_L * j, _L)] = gb[r, pl.ds(_D + _L * j, _L)]
            return _

        lax.fori_loop(0, _CH, body, None)

    def start_wb(t):
        dst0 = out_sin.at[row, pl.ds(off + t * _CH, _CH)]
        dst1 = out_cos.at[row, pl.ds(off + t * _CH, _CH)]
        return (pltpu.async_copy(sbufs[t % _NSP], dst0, w_sem.at[0, t % _NSP]),
                pltpu.async_copy(cbufs[t % _NSP], dst1, w_sem.at[1, t % _NSP]))

    gathers = {t: start_gather(t) for t in range(_NG)}
    wbs = {}
    for t in range(_NCH):
        gathers[t].wait()
        if t >= _NSP:
            for h in wbs[t - _NSP]:
                h.wait()
        deinterleave(t)
        wbs[t] = start_wb(t)
        if t + _NG < _NCH:
            gathers[t + _NG] = start_gather(t + _NG)
    for t in range(max(0, _NCH - _NSP), _NCH):
        for h in wbs[t]:
            h.wait()


def kernel(position, sin_values, cos_values):
    ctab = jnp.concatenate([sin_values, cos_values], axis=1)
    return _gather_rows(position, ctab)


# speedup vs baseline: 1.3282x; 1.0854x over previous
import functools

import jax
import jax.numpy as jnp
from jax import lax
from jax.experimental import pallas as pl
from jax.experimental.pallas import tpu as pltpu
from jax.experimental.pallas import tpu_sc as plsc

_BATCH = 4
_SEQ = 8192
_D = 64
_NC, _NS = 2, 16
_NW = _NC * _NS
_DB = 8
_NSH = 2
_SH = _SEQ // _NSH
_CH = 2048
_NCHB = _SH // _CH
_NTASK = _BATCH * _NCHB
_L = 16

_mesh = plsc.VectorSubcoreMesh(core_axis_name="c", subcore_axis_name="s")


@functools.partial(
    pl.kernel,
    mesh=_mesh,
    out_type=(
        jax.ShapeDtypeStruct((_BATCH, _D, _SEQ), jnp.float32),
        jax.ShapeDtypeStruct((_BATCH, _D, _SEQ), jnp.float32),
    ),
    scratch_types=[
        pltpu.VMEM((_DB, _SEQ), jnp.float32),
        [pltpu.VMEM((_CH,), jnp.int32) for _ in range(2)],
        [pltpu.VMEM((_DB, _CH), jnp.float32) for _ in range(2)],
        pltpu.SemaphoreType.DMA((2,)),
        pltpu.SemaphoreType.DMA((2,)),
    ],
    compiler_params=pltpu.CompilerParams(needs_layout_passes=False),
)
def _gather_t(pos_hbm, sint_hbm, cost_hbm, out_sin, out_cos,
              slab, pbufs, obufs, p_sem, w_sem):
    wid = lax.axis_index("s") * _NC + lax.axis_index("c")
    table = wid // (_NW // 2)
    dblk = (wid % (_NW // 2)) // _NSH
    shalf = wid % _NSH

    @pl.when(table == 0)
    def _():
        pltpu.sync_copy(sint_hbm.at[pl.ds(dblk * _DB, _DB)], slab)

    @pl.when(table == 1)
    def _():
        pltpu.sync_copy(cost_hbm.at[pl.ds(dblk * _DB, _DB)], slab)

    def start_pos(t):
        b, c = t // _NCHB, t % _NCHB
        src = pos_hbm.at[b, pl.ds(shalf * _SH + c * _CH, _CH)]
        return pltpu.async_copy(src, pbufs[t % 2], p_sem.at[t % 2])

    def _wb_dst(out, t):
        b, c = t // _NCHB, t % _NCHB
        return out.at[b, pl.ds(dblk * _DB, _DB),
                      pl.ds(shalf * _SH + c * _CH, _CH)]

    def start_wb(t):
        @pl.when(table == 0)
        def _():
            pltpu.async_copy(obufs[t % 2], _wb_dst(out_sin, t), w_sem.at[t % 2])

        @pl.when(table == 1)
        def _():
            pltpu.async_copy(obufs[t % 2], _wb_dst(out_cos, t), w_sem.at[t % 2])

        return pltpu.make_async_copy(obufs[t % 2], _wb_dst(out_sin, t),
                                     w_sem.at[t % 2])

    def fill(t):
        pb, ob = pbufs[t % 2], obufs[t % 2]

        def body(k, _):
            pvec = pb[pl.ds(k * _L, _L)]
            for d in range(_DB):
                row = jnp.full((_L,), d, jnp.int32)
                ob[d, pl.ds(k * _L, _L)] = plsc.load_gather(slab, [row, pvec])
            return _

        lax.fori_loop(0, _CH // _L, body, None)

    poss = {0: start_pos(0), 1: start_pos(1)}
    wbs = {}
    for t in range(_NTASK):
        poss[t].wait()
        if t >= 2:
            wbs[t - 2].wait()
        fill(t)
        wbs[t] = start_wb(t)
        if t + 2 < _NTASK:
            poss[t + 2] = start_pos(t + 2)
    for t in range(_NTASK - 2, _NTASK):
        wbs[t].wait()


def kernel(position, sin_values, cos_values):
    sin_t, cos_t = _gather_t(position, sin_values.T, cos_values.T)
    return (
        jnp.transpose(sin_t, (0, 2, 1)),
        jnp.transpose(cos_t, (0, 2, 1)),
    )


# speedup vs baseline: 2.2016x; 1.6576x over previous
import functools

import jax
import jax.numpy as jnp
from jax import lax
from jax.experimental import pallas as pl
from jax.experimental.pallas import tpu as pltpu
from jax.experimental.pallas import tpu_sc as plsc

_BATCH = 4
_SEQ = 8192
_D = 64
_NC, _NS = 2, 16
_NW = _NC * _NS
_DB = 8
_NSH = 2
_SH = _SEQ // _NSH
_CH = 2048
_NCHB = _SH // _CH
_NTASK = _BATCH * _NCHB
_L = 16

_mesh = plsc.VectorSubcoreMesh(core_axis_name="c", subcore_axis_name="s")


@functools.partial(
    pl.kernel,
    mesh=_mesh,
    out_type=(
        jax.ShapeDtypeStruct((_BATCH, _D, _SEQ), jnp.float32),
        jax.ShapeDtypeStruct((_BATCH, _D, _SEQ), jnp.float32),
    ),
    scratch_types=[
        pltpu.VMEM((_DB, _SEQ), jnp.float32),
        [pltpu.VMEM((_CH,), jnp.int32) for _ in range(2)],
        [pltpu.VMEM((_DB, _CH), jnp.float32) for _ in range(2)],
        pltpu.SemaphoreType.DMA((2,)),
        pltpu.SemaphoreType.DMA((2,)),
    ],
    compiler_params=pltpu.CompilerParams(needs_layout_passes=False),
)
def _gather_t(pos_hbm, sint_hbm, cost_hbm, out_sin, out_cos,
              slab, pbufs, obufs, p_sem, w_sem):
    wid = lax.axis_index("s") * _NC + lax.axis_index("c")
    table = wid // (_NW // 2)
    dblk = (wid % (_NW // 2)) // _NSH
    shalf = wid % _NSH

    @pl.when(table == 0)
    def _():
        pltpu.sync_copy(sint_hbm.at[pl.ds(dblk * _DB, _DB)], slab)

    @pl.when(table == 1)
    def _():
        pltpu.sync_copy(cost_hbm.at[pl.ds(dblk * _DB, _DB)], slab)

    def start_pos(t):
        b, c = t // _NCHB, t % _NCHB
        src = pos_hbm.at[b, pl.ds(shalf * _SH + c * _CH, _CH)]
        return pltpu.async_copy(src, pbufs[t % 2], p_sem.at[t % 2])

    def _wb_dst(out, t):
        b, c = t // _NCHB, t % _NCHB
        return out.at[b, pl.ds(dblk * _DB, _DB),
                      pl.ds(shalf * _SH + c * _CH, _CH)]

    def start_wb(t):
        @pl.when(table == 0)
        def _():
            pltpu.async_copy(obufs[t % 2], _wb_dst(out_sin, t), w_sem.at[t % 2])

        @pl.when(table == 1)
        def _():
            pltpu.async_copy(obufs[t % 2], _wb_dst(out_cos, t), w_sem.at[t % 2])

        return pltpu.make_async_copy(obufs[t % 2], _wb_dst(out_sin, t),
                                     w_sem.at[t % 2])

    def fill(t):
        pb, ob = pbufs[t % 2], obufs[t % 2]
        rows = [jnp.full((_L,), d, jnp.int32) for d in range(_DB)]

        @plsc.parallel_loop(0, _CH, _L, unroll=4)
        def _(s):
            pvec = pb[pl.ds(s, _L)]
            for d in range(_DB):
                ob[d, pl.ds(s, _L)] = plsc.load_gather(slab, [rows[d], pvec])

    poss = {0: start_pos(0), 1: start_pos(1)}
    wbs = {}
    for t in range(_NTASK):
        poss[t].wait()
        if t >= 2:
            wbs[t - 2].wait()
        fill(t)
        wbs[t] = start_wb(t)
        if t + 2 < _NTASK:
            poss[t + 2] = start_pos(t + 2)
    for t in range(_NTASK - 2, _NTASK):
        wbs[t].wait()


def kernel(position, sin_values, cos_values):
    sin_t, cos_t = _gather_t(position, sin_values.T, cos_values.T)
    return (
        jnp.transpose(sin_t, (0, 2, 1)),
        jnp.transpose(cos_t, (0, 2, 1)),
    )
